# baseline (speedup 1.0000x reference)
import functools

import jax
import jax.numpy as jnp
from jax import lax
from jax.experimental import pallas as pl
from jax.experimental.pallas import tpu as pltpu
from jax.experimental.pallas import tpu_sc as plsc

N = 2000
E = 32000
R2 = 2 * E
BN = 40
NBANDS = N // BN
BROWS = 3 * BN
NCOLS = 3 * N
BSIZE = BROWS * NCOLS
GRP = 128
NS = 16
EPT = E // NS
TCH = BSIZE // NS
ZCH = TCH // 3
ZBUF = 15008

_mesh = plsc.VectorSubcoreMesh(core_axis_name="c", subcore_axis_name="s")


@functools.partial(
    pl.kernel,
    out_type=jax.ShapeDtypeStruct((NCOLS * NCOLS,), jnp.float32),
    mesh=_mesh,
    compiler_params=pltpu.CompilerParams(needs_layout_passes=False),
    scratch_types=[
        pltpu.VMEM((2 * N,), jnp.float32),
        pltpu.VMEM((EPT,), jnp.int32),
        pltpu.VMEM((EPT,), jnp.int32),
        pltpu.VMEM((EPT,), jnp.float32),
        pltpu.VMEM((EPT,), jnp.float32),
        pltpu.VMEM((EPT * 8,), jnp.float32),
        pltpu.VMEM((GRP,), jnp.int32),
        pltpu.VMEM((8 * GRP,), jnp.int32),
        pltpu.VMEM((8 * GRP,), jnp.float32),
        pltpu.VMEM((18 * GRP,), jnp.float32),
        pltpu.VMEM((ZBUF,), jnp.float32),
        pltpu.VMEM((ZBUF,), jnp.float32),
        pltpu.VMEM((NBANDS * 2 * 16,), jnp.int32),
    ] + [pltpu.VMEM((GRP,), jnp.int32) for _ in range(18)]
    + [
        pltpu.VMEM_SHARED((E * 8,), jnp.float32),
        pltpu.VMEM_SHARED((BSIZE,), jnp.float32),
    ],
)
def _assemble(coords, srcs, dsts, emods, avals, perm, meta, out,
              coords_v, esrc, edst, eemod, ea, tabst, permv, eidx8, rows8,
              valv, zrow, bounce, metav, *rest):
    idxrefs = rest[:18]
    tab_s, band_s = rest[18], rest[19]
    t = lax.axis_index("s")
    core = lax.axis_index("c")
    iota = lax.iota(jnp.int32, 16)

    pltpu.sync_copy(coords, coords_v)
    base_e = t * EPT
    pltpu.sync_copy(srcs.at[pl.ds(base_e, EPT)], esrc)
    pltpu.sync_copy(dsts.at[pl.ds(base_e, EPT)], edst)
    pltpu.sync_copy(emods.at[pl.ds(base_e, EPT)], eemod)
    pltpu.sync_copy(avals.at[pl.ds(base_e, EPT)], ea)
    pltpu.sync_copy(meta, metav)

    def p1(i, carry):
        off = i * 16
        s16 = esrc[pl.ds(off, 16)]
        d16 = edst[pl.ds(off, 16)]
        em = eemod[pl.ds(off, 16)]
        aa = ea[pl.ds(off, 16)]
        xs = plsc.load_gather(coords_v, [s16 * 2])
        ys = plsc.load_gather(coords_v, [s16 * 2 + 1])
        xd = plsc.load_gather(coords_v, [d16 * 2])
        yd = plsc.load_gather(coords_v, [d16 * 2 + 1])
        dx = xs - xd
        dy = ys - yd
        l2 = dx * dx + dy * dy
        bits = plsc.bitcast(l2, jnp.int32)
        y = plsc.bitcast(
            jnp.full((16,), 0x5F3759DF, jnp.int32)
            - lax.shift_right_logical(bits, 1),
            jnp.float32,
        )
        h = 0.5 * l2
        y = y * (1.5 - h * y * y)
        y = y * (1.5 - h * y * y)
        y = y * (1.5 - h * y * y)
        lv = l2 * y
        cosv = dx * y
        sinv = -(dy * y)
        kr = em * (aa * aa) * (1.0 / 12.0) * (y * y * y)
        kl = em * aa * y
        kr12 = 12.0 * kr
        ss = sinv * sinv
        cc = cosv * cosv
        scv = sinv * cosv
        pv = kr12 * ss + kl * cc
        rv = kr12 * cc + kl * ss
        qv = scv * (kr12 - kl)
        krl6 = 6.0 * kr * lv
        sv = krl6 * sinv
        cv = krl6 * cosv
        f4 = 4.0 * kr * l2
        rows8x = (off + iota) * 8
        fields = (pv, qv, rv, sv, cv, f4,
                  plsc.bitcast(s16, jnp.float32),
                  plsc.bitcast(d16, jnp.float32))
        for f, v in enumerate(fields):
            plsc.store_scatter(tabst, [rows8x + f], v)
        return carry

    lax.fori_loop(0, EPT // 16, p1, 0)
    pltpu.sync_copy(tabst, tab_s.at[pl.ds(base_e * 8, EPT * 8)])

    def zinit(i, carry):
        zrow[pl.ds(i * 16, 16)] = jnp.zeros((16,), jnp.float32)
        return carry

    lax.fori_loop(0, ZBUF // 16, zinit, 0)
    plsc.subcore_barrier()

    def band_loop(k, carry):
        b = core + 2 * k
        for z in range(3):
            pltpu.sync_copy(zrow.at[pl.ds(0, ZCH)],
                            band_s.at[pl.ds(t * TCH + z * ZCH, ZCH)])
        plsc.subcore_barrier()
        g0 = jnp.max(metav[pl.ds((2 * b) * 16, 16)])
        ng = jnp.max(metav[pl.ds((2 * b + 1) * 16, 16)])
        b80 = b * BN
        boff = b * BSIZE

        def grp_cond(g):
            return g < g0 + ng

        def grp_body(g):
            pltpu.sync_copy(perm.at[pl.ds(g * GRP, GRP)], permv)
            for i in range(8):
                r16 = permv[pl.ds(i * 16, 16)]
                e8 = jnp.where(r16 >= E, r16 - E, r16) * 8
                for j in range(8):
                    eidx8[pl.ds(j * GRP + i * 16, 16)] = e8 + j
            for j in range(8):
                pltpu.sync_copy(
                    tab_s.at[eidx8.at[pl.ds(j * GRP, GRP)]],
                    rows8.at[pl.ds(j * GRP, GRP)])
            for i in range(8):
                fld = [rows8[pl.ds(j * GRP + i * 16, 16)] for j in range(8)]
                pv, qv, rv, sv, cv, f4 = fld[0], fld[1], fld[2], fld[3], fld[4], fld[5]
                s16 = plsc.bitcast(fld[6], jnp.int32)
                d16 = plsc.bitcast(fld[7], jnp.int32)
                r16 = permv[pl.ds(i * 16, 16)]
                isd = r16 >= E
                rn = jnp.where(isd, d16, s16)
                on = jnp.where(isd, s16, d16)
                inb = (rn >= b80) & (rn < b80 + BN)
                m = jnp.where(inb, 1.0, 0.0)
                tsg = jnp.where(isd, -1.0, 1.0)
                pm = pv * m
                qm = qv * m
                rm = rv * m
                sm = sv * tsg * m
                cm = cv * tsg * m
                f4m = f4 * m
                f2m = 0.5 * f4m
                r0 = rn * (3 * NCOLS) - boff
                r1 = r0 + NCOLS
                r2 = r1 + NCOLS
                co = rn * 3
                cb = on * 3
                vals = (pm, qm, -sm, qm, rm, -cm, -sm, -cm, f4m,
                        -pm, -qm, -sm, -qm, -rm, -cm, sm, cm, f2m)
                idxs = (r0 + co, r0 + co + 1, r0 + co + 2,
                        r1 + co, r1 + co + 1, r1 + co + 2,
                        r2 + co, r2 + co + 1, r2 + co + 2,
                        r0 + cb, r0 + cb + 1, r0 + cb + 2,
                        r1 + cb, r1 + cb + 1, r1 + cb + 2,
                        r2 + cb, r2 + cb + 1, r2 + cb + 2)
                for slot in range(18):
                    valv[pl.ds(slot * GRP + i * 16, 16)] = vals[slot]
                    idxrefs[slot][pl.ds(i * 16, 16)] = jnp.clip(
                        idxs[slot], 0, BSIZE - 1)
            for slot in range(18):
                pltpu.sync_copy(valv.at[pl.ds(slot * GRP, GRP)],
                                band_s.at[idxrefs[slot]], add=True)
            return g + NS

        lax.while_loop(grp_cond, grp_body, g0 + t)
        plsc.subcore_barrier()
        outbase = b * BSIZE + t * TCH
        for z in range(3):
            pltpu.sync_copy(band_s.at[pl.ds(t * TCH + z * ZCH, ZCH)],
                            bounce.at[pl.ds(0, ZCH)])
            pltpu.sync_copy(bounce.at[pl.ds(0, ZCH)],
                            out.at[pl.ds(outbase + z * ZCH, ZCH)])
        plsc.subcore_barrier()
        return carry

    lax.fori_loop(0, NBANDS // 2 + (1 - core) * (NBANDS % 2), band_loop, 0)


def kernel(coordinates, edge_index, E_mod, A):
    src = edge_index[0]
    dst = edge_index[1]
    rownode = jnp.concatenate([src, dst])
    band = rownode // BN
    onehot = (band[:, None] == jnp.arange(NBANDS, dtype=jnp.int32)[None, :])
    cum = jnp.cumsum(onehot.astype(jnp.int32), axis=0)
    counts = cum[-1]
    rank = jnp.take_along_axis(cum, band[:, None], axis=1)[:, 0] - 1
    starts = jnp.concatenate(
        [jnp.zeros((1,), jnp.int32), jnp.cumsum(counts)]).astype(jnp.int32)
    pos = starts[band] + rank
    perm = jnp.zeros((R2,), jnp.int32).at[pos].set(
        jnp.arange(R2, dtype=jnp.int32), unique_indices=True,
        mode="promise_in_bounds")
    g0 = starts[:-1] // GRP
    gend = -((-starts[1:]) // GRP)
    meta = jnp.stack([g0, gend - g0], axis=1).reshape(-1)
    meta16 = jnp.broadcast_to(meta[:, None], (NBANDS * 2, 16)).reshape(-1)
    flat = _assemble(coordinates.reshape(-1), src, dst, E_mod, A, perm,
                     meta16.astype(jnp.int32))
    return flat.reshape(NCOLS, NCOLS)


# speedup vs baseline: 1.7127x; 1.7127x over previous
import functools

import jax
import jax.numpy as jnp
from jax import lax
from jax.experimental import pallas as pl
from jax.experimental.pallas import tpu as pltpu
from jax.experimental.pallas import tpu_sc as plsc

N = 2000
E = 32000
R2 = 2 * E
BN = 40
NBANDS = N // BN
BROWS = 3 * BN
NCOLS = 3 * N
BSIZE = BROWS * NCOLS
GRP = 128
NS = 16
EPT = E // NS
TCH = BSIZE // NS
ZCH = TCH // 3
ZBUF = 15008

_mesh = plsc.VectorSubcoreMesh(core_axis_name="c", subcore_axis_name="s")


@functools.partial(
    pl.kernel,
    out_type=jax.ShapeDtypeStruct((NCOLS * NCOLS,), jnp.float32),
    mesh=_mesh,
    compiler_params=pltpu.CompilerParams(needs_layout_passes=False),
    scratch_types=[
        pltpu.VMEM((2 * N,), jnp.float32),
        pltpu.VMEM((EPT,), jnp.int32),
        pltpu.VMEM((EPT,), jnp.int32),
        pltpu.VMEM((EPT,), jnp.float32),
        pltpu.VMEM((EPT,), jnp.float32),
        pltpu.VMEM((EPT * 8,), jnp.float32),
        pltpu.VMEM((GRP,), jnp.int32),
        pltpu.VMEM((8 * GRP,), jnp.int32),
        pltpu.VMEM((8 * GRP,), jnp.float32),
        pltpu.VMEM((18 * GRP,), jnp.float32),
        pltpu.VMEM((ZBUF,), jnp.float32),
        pltpu.VMEM((ZBUF,), jnp.float32),
        pltpu.VMEM((NBANDS * 2 * 16,), jnp.int32),
    ] + [pltpu.VMEM((GRP,), jnp.int32) for _ in range(18)]
    + [
        pltpu.VMEM_SHARED((E * 8,), jnp.float32),
        pltpu.VMEM_SHARED((BSIZE,), jnp.float32),
    ],
)
def _assemble(coords, srcs, dsts, emods, avals, perm, meta, out,
              coords_v, esrc, edst, eemod, ea, tabst, permv, eidx8, rows8,
              valv, zrow, bounce, metav, *rest):
    idxrefs = rest[:18]
    tab_s, band_s = rest[18], rest[19]
    t = lax.axis_index("s")
    core = lax.axis_index("c")
    iota = lax.iota(jnp.int32, 16)

    pltpu.sync_copy(coords, coords_v)
    base_e = t * EPT
    pltpu.sync_copy(srcs.at[pl.ds(base_e, EPT)], esrc)
    pltpu.sync_copy(dsts.at[pl.ds(base_e, EPT)], edst)
    pltpu.sync_copy(emods.at[pl.ds(base_e, EPT)], eemod)
    pltpu.sync_copy(avals.at[pl.ds(base_e, EPT)], ea)
    pltpu.sync_copy(meta, metav)

    def p1(i, carry):
        off = i * 16
        s16 = esrc[pl.ds(off, 16)]
        d16 = edst[pl.ds(off, 16)]
        em = eemod[pl.ds(off, 16)]
        aa = ea[pl.ds(off, 16)]
        xs = plsc.load_gather(coords_v, [s16 * 2])
        ys = plsc.load_gather(coords_v, [s16 * 2 + 1])
        xd = plsc.load_gather(coords_v, [d16 * 2])
        yd = plsc.load_gather(coords_v, [d16 * 2 + 1])
        dx = xs - xd
        dy = ys - yd
        l2 = dx * dx + dy * dy
        bits = plsc.bitcast(l2, jnp.int32)
        y = plsc.bitcast(
            jnp.full((16,), 0x5F3759DF, jnp.int32)
            - lax.shift_right_logical(bits, 1),
            jnp.float32,
        )
        h = 0.5 * l2
        y = y * (1.5 - h * y * y)
        y = y * (1.5 - h * y * y)
        y = y * (1.5 - h * y * y)
        lv = l2 * y
        cosv = dx * y
        sinv = -(dy * y)
        kr = em * (aa * aa) * (1.0 / 12.0) * (y * y * y)
        kl = em * aa * y
        kr12 = 12.0 * kr
        ss = sinv * sinv
        cc = cosv * cosv
        scv = sinv * cosv
        pv = kr12 * ss + kl * cc
        rv = kr12 * cc + kl * ss
        qv = scv * (kr12 - kl)
        krl6 = 6.0 * kr * lv
        sv = krl6 * sinv
        cv = krl6 * cosv
        f4 = 4.0 * kr * l2
        rows8x = (off + iota) * 8
        fields = (pv, qv, rv, sv, cv, f4,
                  plsc.bitcast(s16, jnp.float32),
                  plsc.bitcast(d16, jnp.float32))
        for f, v in enumerate(fields):
            plsc.store_scatter(tabst, [rows8x + f], v)
        return carry

    lax.fori_loop(0, EPT // 16, p1, 0)
    pltpu.sync_copy(tabst, tab_s.at[pl.ds(base_e * 8, EPT * 8)])

    def zinit(i, carry):
        zrow[pl.ds(i * 16, 16)] = jnp.zeros((16,), jnp.float32)
        return carry

    lax.fori_loop(0, ZBUF // 16, zinit, 0)
    plsc.subcore_barrier()

    def band_loop(k, carry):
        b = core + 2 * k
        for z in range(3):
            pltpu.sync_copy(zrow.at[pl.ds(0, ZCH)],
                            band_s.at[pl.ds(t * TCH + z * ZCH, ZCH)])
        plsc.subcore_barrier()
        g0 = jnp.max(metav[pl.ds((2 * b) * 16, 16)])
        ng = jnp.max(metav[pl.ds((2 * b + 1) * 16, 16)])
        b80 = b * BN
        boff = b * BSIZE

        def grp_cond(g):
            return g < g0 + ng

        def grp_body(g):
            pltpu.sync_copy(perm.at[pl.ds(g * GRP, GRP)], permv)
            for i in range(8):
                r16 = permv[pl.ds(i * 16, 16)]
                e8 = jnp.where(r16 >= E, r16 - E, r16) * 8
                for j in range(8):
                    eidx8[pl.ds(j * GRP + i * 16, 16)] = e8 + j
            for j in range(8):
                pltpu.sync_copy(
                    tab_s.at[eidx8.at[pl.ds(j * GRP, GRP)]],
                    rows8.at[pl.ds(j * GRP, GRP)])
            for i in range(8):
                fld = [rows8[pl.ds(j * GRP + i * 16, 16)] for j in range(8)]
                pv, qv, rv, sv, cv, f4 = fld[0], fld[1], fld[2], fld[3], fld[4], fld[5]
                s16 = plsc.bitcast(fld[6], jnp.int32)
                d16 = plsc.bitcast(fld[7], jnp.int32)
                r16 = permv[pl.ds(i * 16, 16)]
                isd = r16 >= E
                rn = jnp.where(isd, d16, s16)
                on = jnp.where(isd, s16, d16)
                inb = (rn >= b80) & (rn < b80 + BN)
                m = jnp.where(inb, 1.0, 0.0)
                tsg = jnp.where(isd, -1.0, 1.0)
                pm = pv * m
                qm = qv * m
                rm = rv * m
                sm = sv * tsg * m
                cm = cv * tsg * m
                f4m = f4 * m
                f2m = 0.5 * f4m
                r0 = rn * (3 * NCOLS) - boff
                r1 = r0 + NCOLS
                r2 = r1 + NCOLS
                co = rn * 3
                cb = on * 3
                vals = (pm, qm, -sm, qm, rm, -cm, -sm, -cm, f4m,
                        -pm, -qm, -sm, -qm, -rm, -cm, sm, cm, f2m)
                idxs = (r0 + co, r0 + co + 1, r0 + co + 2,
                        r1 + co, r1 + co + 1, r1 + co + 2,
                        r2 + co, r2 + co + 1, r2 + co + 2,
                        r0 + cb, r0 + cb + 1, r0 + cb + 2,
                        r1 + cb, r1 + cb + 1, r1 + cb + 2,
                        r2 + cb, r2 + cb + 1, r2 + cb + 2)
                for slot in range(18):
                    valv[pl.ds(slot * GRP + i * 16, 16)] = vals[slot]
                    idxrefs[slot][pl.ds(i * 16, 16)] = jnp.clip(
                        idxs[slot], 0, BSIZE - 1)
            for slot in range(18):
                pltpu.sync_copy(valv.at[pl.ds(slot * GRP, GRP)],
                                band_s.at[idxrefs[slot]], add=True)
            return g + NS

        lax.while_loop(grp_cond, grp_body, g0 + t)
        plsc.subcore_barrier()
        outbase = b * BSIZE + t * TCH
        for z in range(3):
            pltpu.sync_copy(band_s.at[pl.ds(t * TCH + z * ZCH, ZCH)],
                            bounce.at[pl.ds(0, ZCH)])
            pltpu.sync_copy(bounce.at[pl.ds(0, ZCH)],
                            out.at[pl.ds(outbase + z * ZCH, ZCH)])
        plsc.subcore_barrier()
        return carry

    lax.fori_loop(0, NBANDS // 2 + (1 - core) * (NBANDS % 2), band_loop, 0)


def kernel(coordinates, edge_index, E_mod, A):
    src = edge_index[0]
    dst = edge_index[1]
    rownode = jnp.concatenate([src, dst])
    band = rownode // BN
    perm = jnp.argsort(band, stable=True).astype(jnp.int32)
    counts = jnp.sum(
        (band[:, None] == jnp.arange(NBANDS, dtype=jnp.int32)[None, :])
        .astype(jnp.int32), axis=0)
    starts = jnp.concatenate(
        [jnp.zeros((1,), jnp.int32), jnp.cumsum(counts)]).astype(jnp.int32)
    g0 = starts[:-1] // GRP
    gend = -((-starts[1:]) // GRP)
    meta = jnp.stack([g0, gend - g0], axis=1).reshape(-1)
    meta16 = jnp.broadcast_to(meta[:, None], (NBANDS * 2, 16)).reshape(-1)
    flat = _assemble(coordinates.reshape(-1), src, dst, E_mod, A, perm,
                     meta16.astype(jnp.int32))
    return flat.reshape(NCOLS, NCOLS)


# speedup vs baseline: 1.8020x; 1.0521x over previous
import functools

import jax
import jax.numpy as jnp
from jax import lax
from jax.experimental import pallas as pl
from jax.experimental.pallas import tpu as pltpu
from jax.experimental.pallas import tpu_sc as plsc

N = 2000
E = 32000
R2 = 2 * E
BN = 40
NBANDS = N // BN
BROWS = 3 * BN
NCOLS = 3 * N
BSIZE = BROWS * NCOLS
GRP = 128
NS = 16
EPT = E // NS
TCH = BSIZE // NS
ZCH = TCH // 3
ZBUF = 15008

_mesh = plsc.VectorSubcoreMesh(core_axis_name="c", subcore_axis_name="s")


@functools.partial(
    pl.kernel,
    out_type=jax.ShapeDtypeStruct((NCOLS * NCOLS,), jnp.float32),
    mesh=_mesh,
    compiler_params=pltpu.CompilerParams(needs_layout_passes=False),
    scratch_types=[
        pltpu.VMEM((2 * N,), jnp.float32),
        pltpu.VMEM((EPT,), jnp.int32),
        pltpu.VMEM((EPT,), jnp.int32),
        pltpu.VMEM((EPT,), jnp.float32),
        pltpu.VMEM((EPT,), jnp.float32),
        pltpu.VMEM((EPT * 8,), jnp.float32),
        pltpu.VMEM((GRP,), jnp.int32),
        pltpu.VMEM((8 * GRP,), jnp.int32),
        pltpu.VMEM((8 * GRP,), jnp.float32),
        pltpu.VMEM((18 * GRP,), jnp.float32),
        pltpu.VMEM((ZBUF,), jnp.float32),
        pltpu.VMEM((ZBUF,), jnp.float32),
        pltpu.VMEM((NBANDS * 2 * 16,), jnp.int32),
    ] + [pltpu.VMEM((GRP,), jnp.int32) for _ in range(18)]
    + [
        pltpu.VMEM_SHARED((E * 8,), jnp.float32),
        pltpu.VMEM_SHARED((BSIZE,), jnp.float32),
        pltpu.SemaphoreType.DMA,
        pltpu.SemaphoreType.DMA,
    ],
)
def _assemble(coords, srcs, dsts, emods, avals, perm, meta, out,
              coords_v, esrc, edst, eemod, ea, tabst, permv, eidx8, rows8,
              valv, zrow, bounce, metav, *rest):
    idxrefs = rest[:18]
    tab_s, band_s = rest[18], rest[19]
    sem_g, sem_s = rest[20], rest[21]
    t = lax.axis_index("s")
    core = lax.axis_index("c")
    iota = lax.iota(jnp.int32, 16)

    pltpu.sync_copy(coords, coords_v)
    base_e = t * EPT
    pltpu.sync_copy(srcs.at[pl.ds(base_e, EPT)], esrc)
    pltpu.sync_copy(dsts.at[pl.ds(base_e, EPT)], edst)
    pltpu.sync_copy(emods.at[pl.ds(base_e, EPT)], eemod)
    pltpu.sync_copy(avals.at[pl.ds(base_e, EPT)], ea)
    pltpu.sync_copy(meta, metav)

    def p1(i, carry):
        off = i * 16
        s16 = esrc[pl.ds(off, 16)]
        d16 = edst[pl.ds(off, 16)]
        em = eemod[pl.ds(off, 16)]
        aa = ea[pl.ds(off, 16)]
        xs = plsc.load_gather(coords_v, [s16 * 2])
        ys = plsc.load_gather(coords_v, [s16 * 2 + 1])
        xd = plsc.load_gather(coords_v, [d16 * 2])
        yd = plsc.load_gather(coords_v, [d16 * 2 + 1])
        dx = xs - xd
        dy = ys - yd
        l2 = dx * dx + dy * dy
        bits = plsc.bitcast(l2, jnp.int32)
        y = plsc.bitcast(
            jnp.full((16,), 0x5F3759DF, jnp.int32)
            - lax.shift_right_logical(bits, 1),
            jnp.float32,
        )
        h = 0.5 * l2
        y = y * (1.5 - h * y * y)
        y = y * (1.5 - h * y * y)
        y = y * (1.5 - h * y * y)
        lv = l2 * y
        cosv = dx * y
        sinv = -(dy * y)
        kr = em * (aa * aa) * (1.0 / 12.0) * (y * y * y)
        kl = em * aa * y
        kr12 = 12.0 * kr
        ss = sinv * sinv
        cc = cosv * cosv
        scv = sinv * cosv
        pv = kr12 * ss + kl * cc
        rv = kr12 * cc + kl * ss
        qv = scv * (kr12 - kl)
        krl6 = 6.0 * kr * lv
        sv = krl6 * sinv
        cv = krl6 * cosv
        f4 = 4.0 * kr * l2
        rows8x = (off + iota) * 8
        fields = (pv, qv, rv, sv, cv, f4,
                  plsc.bitcast(s16, jnp.float32),
                  plsc.bitcast(d16, jnp.float32))
        for f, v in enumerate(fields):
            plsc.store_scatter(tabst, [rows8x + f], v)
        return carry

    lax.fori_loop(0, EPT // 16, p1, 0)
    pltpu.sync_copy(tabst, tab_s.at[pl.ds(base_e * 8, EPT * 8)])

    def zinit(i, carry):
        zrow[pl.ds(i * 16, 16)] = jnp.zeros((16,), jnp.float32)
        return carry

    lax.fori_loop(0, ZBUF // 16, zinit, 0)
    plsc.subcore_barrier()

    def band_loop(k, carry):
        b = core + 2 * k
        for z in range(3):
            pltpu.sync_copy(zrow.at[pl.ds(0, ZCH)],
                            band_s.at[pl.ds(t * TCH + z * ZCH, ZCH)])
        plsc.subcore_barrier()
        g0 = jnp.max(metav[pl.ds((2 * b) * 16, 16)])
        ng = jnp.max(metav[pl.ds((2 * b + 1) * 16, 16)])
        b80 = b * BN
        boff = b * BSIZE

        def grp_cond(g):
            return g < g0 + ng

        def grp_body(g):
            pltpu.sync_copy(perm.at[pl.ds(g * GRP, GRP)], permv)
            for i in range(8):
                r16 = permv[pl.ds(i * 16, 16)]
                e8 = jnp.where(r16 >= E, r16 - E, r16) * 8
                for j in range(8):
                    eidx8[pl.ds(j * GRP + i * 16, 16)] = e8 + j
            gds = [
                pltpu.async_copy(
                    tab_s.at[eidx8.at[pl.ds(j * GRP, GRP)]],
                    rows8.at[pl.ds(j * GRP, GRP)], sem_g)
                for j in range(8)
            ]
            for gd in gds:
                gd.wait()
            for i in range(8):
                fld = [rows8[pl.ds(j * GRP + i * 16, 16)] for j in range(8)]
                pv, qv, rv, sv, cv, f4 = fld[0], fld[1], fld[2], fld[3], fld[4], fld[5]
                s16 = plsc.bitcast(fld[6], jnp.int32)
                d16 = plsc.bitcast(fld[7], jnp.int32)
                r16 = permv[pl.ds(i * 16, 16)]
                isd = r16 >= E
                rn = jnp.where(isd, d16, s16)
                on = jnp.where(isd, s16, d16)
                inb = (rn >= b80) & (rn < b80 + BN)
                m = jnp.where(inb, 1.0, 0.0)
                tsg = jnp.where(isd, -1.0, 1.0)
                pm = pv * m
                qm = qv * m
                rm = rv * m
                sm = sv * tsg * m
                cm = cv * tsg * m
                f4m = f4 * m
                f2m = 0.5 * f4m
                r0 = rn * (3 * NCOLS) - boff
                r1 = r0 + NCOLS
                r2 = r1 + NCOLS
                co = rn * 3
                cb = on * 3
                vals = (pm, qm, -sm, qm, rm, -cm, -sm, -cm, f4m,
                        -pm, -qm, -sm, -qm, -rm, -cm, sm, cm, f2m)
                idxs = (r0 + co, r0 + co + 1, r0 + co + 2,
                        r1 + co, r1 + co + 1, r1 + co + 2,
                        r2 + co, r2 + co + 1, r2 + co + 2,
                        r0 + cb, r0 + cb + 1, r0 + cb + 2,
                        r1 + cb, r1 + cb + 1, r1 + cb + 2,
                        r2 + cb, r2 + cb + 1, r2 + cb + 2)
                for slot in range(18):
                    valv[pl.ds(slot * GRP + i * 16, 16)] = vals[slot]
                    idxrefs[slot][pl.ds(i * 16, 16)] = jnp.clip(
                        idxs[slot], 0, BSIZE - 1)
            sds = [
                pltpu.async_copy(valv.at[pl.ds(slot * GRP, GRP)],
                                 band_s.at[idxrefs[slot]], sem_s, add=True)
                for slot in range(18)
            ]
            for sd in sds:
                sd.wait()
            return g + NS

        lax.while_loop(grp_cond, grp_body, g0 + t)
        plsc.subcore_barrier()
        outbase = b * BSIZE + t * TCH
        for z in range(3):
            pltpu.sync_copy(band_s.at[pl.ds(t * TCH + z * ZCH, ZCH)],
                            bounce.at[pl.ds(0, ZCH)])
            pltpu.sync_copy(bounce.at[pl.ds(0, ZCH)],
                            out.at[pl.ds(outbase + z * ZCH, ZCH)])
        plsc.subcore_barrier()
        return carry

    lax.fori_loop(0, NBANDS // 2 + (1 - core) * (NBANDS % 2), band_loop, 0)


def kernel(coordinates, edge_index, E_mod, A):
    src = edge_index[0]
    dst = edge_index[1]
    rownode = jnp.concatenate([src, dst])
    band = rownode // BN
    perm = jnp.argsort(band, stable=True).astype(jnp.int32)
    counts = jnp.sum(
        (band[:, None] == jnp.arange(NBANDS, dtype=jnp.int32)[None, :])
        .astype(jnp.int32), axis=0)
    starts = jnp.concatenate(
        [jnp.zeros((1,), jnp.int32), jnp.cumsum(counts)]).astype(jnp.int32)
    g0 = starts[:-1] // GRP
    gend = -((-starts[1:]) // GRP)
    meta = jnp.stack([g0, gend - g0], axis=1).reshape(-1)
    meta16 = jnp.broadcast_to(meta[:, None], (NBANDS * 2, 16)).reshape(-1)
    flat = _assemble(coordinates.reshape(-1), src, dst, E_mod, A, perm,
                     meta16.astype(jnp.int32))
    return flat.reshape(NCOLS, NCOLS)


# speedup vs baseline: 1.8286x; 1.0148x over previous
import functools

import jax
import jax.numpy as jnp
from jax import lax
from jax.experimental import pallas as pl
from jax.experimental.pallas import tpu as pltpu
from jax.experimental.pallas import tpu_sc as plsc

N = 2000
E = 32000
R2 = 2 * E
BN = 40
NBANDS = N // BN
BROWS = 3 * BN
NCOLS = 3 * N
BSIZE = BROWS * NCOLS
GRP = 128
NS = 16
EPT = E // NS
TCH = BSIZE // NS
ZCH = TCH // 3
ZBUF = 15008

_mesh = plsc.VectorSubcoreMesh(core_axis_name="c", subcore_axis_name="s")


@functools.partial(
    pl.kernel,
    out_type=jax.ShapeDtypeStruct((NCOLS * NCOLS,), jnp.float32),
    mesh=_mesh,
    compiler_params=pltpu.CompilerParams(needs_layout_passes=False),
    scratch_types=[
        pltpu.VMEM((2 * N,), jnp.float32),
        pltpu.VMEM((EPT,), jnp.int32),
        pltpu.VMEM((EPT,), jnp.int32),
        pltpu.VMEM((EPT,), jnp.float32),
        pltpu.VMEM((EPT,), jnp.float32),
        pltpu.VMEM((EPT * 8,), jnp.float32),
        pltpu.VMEM((GRP,), jnp.int32),
        pltpu.VMEM((8 * GRP,), jnp.int32),
        pltpu.VMEM((8 * GRP,), jnp.float32),
        pltpu.VMEM((18 * GRP,), jnp.float32),
        pltpu.VMEM((ZBUF,), jnp.float32),
        pltpu.VMEM((ZBUF,), jnp.float32),
        pltpu.VMEM((NBANDS * 2 * 16,), jnp.int32),
    ] + [pltpu.VMEM((GRP,), jnp.int32) for _ in range(18)]
    + [
        pltpu.VMEM_SHARED((E * 8,), jnp.float32),
        pltpu.VMEM_SHARED((BSIZE,), jnp.float32),
        pltpu.SemaphoreType.DMA,
        pltpu.SemaphoreType.DMA,
    ],
)
def _assemble(coords, srcs, dsts, emods, avals, perm, meta, out,
              coords_v, esrc, edst, eemod, ea, tabst, permv, eidx8, rows8,
              valv, zrow, bounce, metav, *rest):
    idxrefs = rest[:18]
    tab_s, band_s = rest[18], rest[19]
    sem_g, sem_s = rest[20], rest[21]
    t = lax.axis_index("s")
    core = lax.axis_index("c")
    iota = lax.iota(jnp.int32, 16)

    pltpu.sync_copy(coords, coords_v)
    base_e = t * EPT
    pltpu.sync_copy(srcs.at[pl.ds(base_e, EPT)], esrc)
    pltpu.sync_copy(dsts.at[pl.ds(base_e, EPT)], edst)
    pltpu.sync_copy(emods.at[pl.ds(base_e, EPT)], eemod)
    pltpu.sync_copy(avals.at[pl.ds(base_e, EPT)], ea)
    pltpu.sync_copy(meta, metav)

    def p1(i, carry):
        off = i * 16
        s16 = esrc[pl.ds(off, 16)]
        d16 = edst[pl.ds(off, 16)]
        em = eemod[pl.ds(off, 16)]
        aa = ea[pl.ds(off, 16)]
        xs = plsc.load_gather(coords_v, [s16 * 2])
        ys = plsc.load_gather(coords_v, [s16 * 2 + 1])
        xd = plsc.load_gather(coords_v, [d16 * 2])
        yd = plsc.load_gather(coords_v, [d16 * 2 + 1])
        dx = xs - xd
        dy = ys - yd
        l2 = dx * dx + dy * dy
        bits = plsc.bitcast(l2, jnp.int32)
        y = plsc.bitcast(
            jnp.full((16,), 0x5F3759DF, jnp.int32)
            - lax.shift_right_logical(bits, 1),
            jnp.float32,
        )
        h = 0.5 * l2
        y = y * (1.5 - h * y * y)
        y = y * (1.5 - h * y * y)
        y = y * (1.5 - h * y * y)
        lv = l2 * y
        cosv = dx * y
        sinv = -(dy * y)
        kr = em * (aa * aa) * (1.0 / 12.0) * (y * y * y)
        kl = em * aa * y
        kr12 = 12.0 * kr
        ss = sinv * sinv
        cc = cosv * cosv
        scv = sinv * cosv
        pv = kr12 * ss + kl * cc
        rv = kr12 * cc + kl * ss
        qv = scv * (kr12 - kl)
        krl6 = 6.0 * kr * lv
        sv = krl6 * sinv
        cv = krl6 * cosv
        f4 = 4.0 * kr * l2
        rows8x = (off + iota) * 8
        fields = (pv, qv, rv, sv, cv, f4,
                  plsc.bitcast(s16, jnp.float32),
                  plsc.bitcast(d16, jnp.float32))
        for f, v in enumerate(fields):
            plsc.store_scatter(tabst, [rows8x + f], v)
        return carry

    lax.fori_loop(0, EPT // 16, p1, 0)
    pltpu.sync_copy(tabst, tab_s.at[pl.ds(base_e * 8, EPT * 8)])

    def zinit(i, carry):
        zrow[pl.ds(i * 16, 16)] = jnp.zeros((16,), jnp.float32)
        return carry

    lax.fori_loop(0, ZBUF // 16, zinit, 0)
    plsc.subcore_barrier()

    def band_loop(k, carry):
        b = core + 2 * k
        for z in range(3):
            pltpu.sync_copy(zrow.at[pl.ds(0, ZCH)],
                            band_s.at[pl.ds(t * TCH + z * ZCH, ZCH)])
        plsc.subcore_barrier()
        g0 = jnp.max(metav[pl.ds((2 * b) * 16, 16)])
        ng = jnp.max(metav[pl.ds((2 * b + 1) * 16, 16)])
        b80 = b * BN
        boff = b * BSIZE

        def grp_cond(g):
            return g < g0 + ng

        def grp_body(g):
            pltpu.sync_copy(perm.at[pl.ds(g * GRP, GRP)], permv)
            for i in range(8):
                r16 = permv[pl.ds(i * 16, 16)]
                e8 = jnp.where(r16 >= E, r16 - E, r16) * 8
                for j in range(8):
                    eidx8[pl.ds(j * GRP + i * 16, 16)] = e8 + j
            gds = [
                pltpu.async_copy(
                    tab_s.at[eidx8.at[pl.ds(j * GRP, GRP)]],
                    rows8.at[pl.ds(j * GRP, GRP)], sem_g)
                for j in range(8)
            ]
            for gd in gds:
                gd.wait()
            for i in range(8):
                fld = [rows8[pl.ds(j * GRP + i * 16, 16)] for j in range(8)]
                pv, qv, rv, sv, cv, f4 = fld[0], fld[1], fld[2], fld[3], fld[4], fld[5]
                s16 = plsc.bitcast(fld[6], jnp.int32)
                d16 = plsc.bitcast(fld[7], jnp.int32)
                r16 = permv[pl.ds(i * 16, 16)]
                isd = r16 >= E
                rn = jnp.where(isd, d16, s16)
                on = jnp.where(isd, s16, d16)
                inb = (rn >= b80) & (rn < b80 + BN)
                m = jnp.where(inb, 1.0, 0.0)
                tsg = jnp.where(isd, -1.0, 1.0)
                pm = pv * m
                qm = qv * m
                rm = rv * m
                sm = sv * tsg * m
                cm = cv * tsg * m
                f4m = f4 * m
                f2m = 0.5 * f4m
                r0 = rn * (3 * NCOLS) - boff
                r1 = r0 + NCOLS
                r2 = r1 + NCOLS
                co = rn * 3
                cb = on * 3
                vals = (pm, qm, -sm, qm, rm, -cm, -sm, -cm, f4m,
                        -pm, -qm, -sm, -qm, -rm, -cm, sm, cm, f2m)
                idxs = (r0 + co, r0 + co + 1, r0 + co + 2,
                        r1 + co, r1 + co + 1, r1 + co + 2,
                        r2 + co, r2 + co + 1, r2 + co + 2,
                        r0 + cb, r0 + cb + 1, r0 + cb + 2,
                        r1 + cb, r1 + cb + 1, r1 + cb + 2,
                        r2 + cb, r2 + cb + 1, r2 + cb + 2)
                for slot in range(18):
                    valv[pl.ds(slot * GRP + i * 16, 16)] = vals[slot]
                    idxrefs[slot][pl.ds(i * 16, 16)] = jnp.clip(
                        idxs[slot], 0, BSIZE - 1)
            sds = [
                pltpu.async_copy(valv.at[pl.ds(slot * GRP, GRP)],
                                 band_s.at[idxrefs[slot]], sem_s, add=True)
                for slot in range(18)
            ]
            for sd in sds:
                sd.wait()
            return g + NS

        lax.while_loop(grp_cond, grp_body, g0 + t)
        plsc.subcore_barrier()
        outbase = b * BSIZE + t * TCH
        for z in range(3):
            pltpu.sync_copy(band_s.at[pl.ds(t * TCH + z * ZCH, ZCH)],
                            bounce.at[pl.ds(0, ZCH)])
            pltpu.sync_copy(bounce.at[pl.ds(0, ZCH)],
                            out.at[pl.ds(outbase + z * ZCH, ZCH)])
        plsc.subcore_barrier()
        return carry

    lax.fori_loop(0, NBANDS // 2 + (1 - core) * (NBANDS % 2), band_loop, 0)


def kernel(coordinates, edge_index, E_mod, A):
    src = edge_index[0]
    dst = edge_index[1]
    rownode = jnp.concatenate([src, dst])
    band = rownode // BN
    perm = jnp.argsort(band, stable=False).astype(jnp.int32)
    counts = jnp.sum(
        (band[:, None] == jnp.arange(NBANDS, dtype=jnp.int32)[None, :])
        .astype(jnp.int32), axis=0)
    starts = jnp.concatenate(
        [jnp.zeros((1,), jnp.int32), jnp.cumsum(counts)]).astype(jnp.int32)
    g0 = starts[:-1] // GRP
    gend = -((-starts[1:]) // GRP)
    meta = jnp.stack([g0, gend - g0], axis=1).reshape(-1)
    meta16 = jnp.broadcast_to(meta[:, None], (NBANDS * 2, 16)).reshape(-1)
    flat = _assemble(coordinates.reshape(-1), src, dst, E_mod, A, perm,
                     meta16.astype(jnp.int32))
    return flat.reshape(NCOLS, NCOLS)
